# baseline (speedup 1.0000x reference)
import functools

import jax
import jax.numpy as jnp
from jax import lax
from jax.experimental import pallas as pl
from jax.experimental.pallas import tpu as pltpu
from jax.experimental.pallas import tpu_sc as plsc

_N = 10000
_D = 256
_COL = 1
_L = 16
_NT = 16
_BPW = 640
_G = _BPW // _L

_mesh = plsc.VectorSubcoreMesh(
    core_axis_name="c", subcore_axis_name="s", num_cores=1
)


@functools.partial(
    pl.kernel,
    mesh=_mesh,
    out_type=jax.ShapeDtypeStruct((2, _NT, _L), jnp.float32),
    scratch_types=[
        pltpu.VMEM((_BPW,), jnp.int32),
        pltpu.VMEM((_BPW,), jnp.float32),
        pltpu.VMEM((_L,), jnp.float32),
        pltpu.VMEM((_NT, _L), jnp.float32),
        pltpu.VMEM((_L,), jnp.float32),
        pltpu.SemaphoreType.DMA,
    ],
)
def _col_sum_sc(x_hbm, out_hbm, idx_v, vals_v, partial_v, all_v, out_v, sem):
    sid = lax.axis_index("s")
    base = sid * _BPW
    lane = lax.iota(jnp.int32, _L)

    def build(g, carry):
        gi = base + g * _L + lane
        gi = jnp.minimum(gi, _N - 1)
        idx_v[pl.ds(g * _L, _L)] = gi * _D + _COL
        return carry

    lax.fori_loop(0, _G, build, 0)

    pltpu.async_copy(x_hbm.at[idx_v], vals_v, sem).wait()

    def acc_body(g, acc):
        gi = base + g * _L + lane
        v = vals_v[pl.ds(g * _L, _L)]
        return acc + jnp.where(gi < _N, v, 0.0)

    acc = lax.fori_loop(0, _G, acc_body, jnp.zeros((_L,), jnp.float32))
    partial_v[...] = acc
    pltpu.sync_copy(partial_v, out_hbm.at[0, sid])
    plsc.subcore_barrier()

    @pl.when(sid == 0)
    def _():
        pltpu.sync_copy(out_hbm.at[0], all_v)
        tot_v = all_v[0]
        for t in range(1, _NT):
            tot_v = tot_v + all_v[t]
        tot = tot_v[0]
        for j in range(1, _L):
            tot = tot + tot_v[j]
        i = lax.bitcast_convert_type(tot, jnp.int32)
        i = 0x5F3759DF - lax.shift_right_logical(i, 1)
        y = lax.bitcast_convert_type(i, jnp.float32)
        y = y * (1.5 - 0.5 * tot * y * y)
        y = y * (1.5 - 0.5 * tot * y * y)
        y = y * (1.5 - 0.5 * tot * y * y)
        r = jnp.where(tot > 0.0, tot * y, 0.0)
        out_v[...] = jnp.full((_L,), r, jnp.float32)
        pltpu.sync_copy(out_v, out_hbm.at[1, 0])


def kernel(node_attr, edgeij_pair, edge_attr, g, batch):
    flat = node_attr.reshape(_N * _D)
    out = _col_sum_sc(flat)
    return out[1, 0, 0]
